# baseline (speedup 1.0000x reference)
import functools

import jax
import jax.numpy as jnp
from jax import lax
from jax.experimental import pallas as pl
from jax.experimental.pallas import tpu as pltpu
from jax.experimental.pallas import tpu_sc as plsc


def _emb_call(B, V, D, NC, NS, C):
    NW = NC * NS
    b_per_w = B // NW
    n_chunks = b_per_w // C
    mesh = plsc.VectorSubcoreMesh(core_axis_name="c", subcore_axis_name="s",
                                  num_cores=NC, num_subcores=NS)

    @functools.partial(
        pl.kernel,
        mesh=mesh,
        out_type=jax.ShapeDtypeStruct((B, D), jnp.float32),
        scratch_types=[
            pltpu.VMEM((C,), jnp.int32),
            pltpu.VMEM((C, D), jnp.float32),
            pltpu.SemaphoreType.DMA,
        ],
        compiler_params=pltpu.CompilerParams(use_tc_tiling_on_sc=False),
    )
    def emb(idx_hbm, table_hbm, out_hbm, idx_v, rows_v, sem):
        wid = lax.axis_index("s") * NC + lax.axis_index("c")
        base = wid * b_per_w

        def body(i, carry):
            off = base + i * C
            pltpu.sync_copy(idx_hbm.at[pl.ds(off, C)], idx_v)
            pltpu.async_copy(table_hbm.at[idx_v], rows_v, sem).wait()
            pltpu.sync_copy(rows_v, out_hbm.at[pl.ds(off, C)])
            return carry

        lax.fori_loop(0, n_chunks, body, 0)

    return emb


def kernel(x, table):
    Bx, S = x.shape
    V, D = table.shape
    B = Bx * S
    idx = x.reshape(B).astype(jnp.int32)
    info = plsc.get_sparse_core_info()
    emb = _emb_call(B, V, D, info.num_cores, info.num_subcores, C=1600)
    out = emb(idx, table)
    return out.reshape(Bx, S, D)


# speedup vs baseline: 1.0043x; 1.0043x over previous
import functools

import jax
import jax.numpy as jnp
from jax import lax
from jax.experimental import pallas as pl
from jax.experimental.pallas import tpu as pltpu
from jax.experimental.pallas import tpu_sc as plsc

_NBUF = 4


def _emb_call(B, V, D, NC, NS, C):
    NW = NC * NS
    b_per_w = B // NW
    n_chunks = b_per_w // C
    n_outer = n_chunks // _NBUF
    mesh = plsc.VectorSubcoreMesh(core_axis_name="c", subcore_axis_name="s",
                                  num_cores=NC, num_subcores=NS)

    @functools.partial(
        pl.kernel,
        mesh=mesh,
        out_type=jax.ShapeDtypeStruct((B, D), jnp.float32),
        scratch_types=[
            pltpu.VMEM((n_chunks, C), jnp.int32),
        ] + [pltpu.VMEM((C, D), jnp.float32) for _ in range(_NBUF)]
          + [pltpu.SemaphoreType.DMA for _ in range(2 * _NBUF)],
        compiler_params=pltpu.CompilerParams(use_tc_tiling_on_sc=False),
    )
    def emb(idx_hbm, table_hbm, out_hbm, idx_v, *rest):
        bufs = rest[:_NBUF]
        sem_g = rest[_NBUF:2 * _NBUF]
        sem_w = rest[2 * _NBUF:]
        wid = lax.axis_index("s") * NC + lax.axis_index("c")
        base = wid * b_per_w

        pltpu.sync_copy(idx_hbm.at[wid], idx_v)

        def gather_start(i, b):
            pltpu.async_copy(table_hbm.at[idx_v.at[i]], bufs[b], sem_g[b])

        def gather_wait(b):
            pltpu.make_async_copy(table_hbm.at[idx_v.at[0]], bufs[b],
                                  sem_g[b]).wait()

        def wb_start(i, b):
            pltpu.async_copy(bufs[b], out_hbm.at[pl.ds(base + i * C, C)],
                             sem_w[b])

        def wb_wait(b):
            pltpu.make_async_copy(bufs[b], out_hbm.at[pl.ds(base, C)],
                                  sem_w[b]).wait()

        for b in range(_NBUF):
            gather_start(b, b)

        def outer(g, carry):
            for b in range(_NBUF):
                i = g * _NBUF + b
                gather_wait(b)
                wb_start(i, b)
                wb_wait(b)
                gather_start(i + _NBUF, b)
            return carry

        lax.fori_loop(0, n_outer - 1, outer, 0)

        for b in range(_NBUF):
            i = (n_outer - 1) * _NBUF + b
            gather_wait(b)
            wb_start(i, b)
            wb_wait(b)

    return emb


def kernel(x, table):
    Bx, S = x.shape
    V, D = table.shape
    B = Bx * S
    info = plsc.get_sparse_core_info()
    NW = info.num_cores * info.num_subcores
    C = 400
    idx = x.reshape(NW, (B // NW) // C, C).astype(jnp.int32)
    emb = _emb_call(B, V, D, info.num_cores, info.num_subcores, C)
    out = emb(idx, table)
    return out.reshape(Bx, S, D)


# speedup vs baseline: 1.2255x; 1.2203x over previous
import functools

import jax
import jax.numpy as jnp
from jax import lax
from jax.experimental import pallas as pl
from jax.experimental.pallas import tpu as pltpu
from jax.experimental.pallas import tpu_sc as plsc

_NBUF = 2


def _emb_call(B, V, D, NC, NS, C):
    NW = NC * NS
    b_per_w = B // NW
    n_chunks = b_per_w // C
    n_outer = n_chunks // _NBUF
    mesh = plsc.VectorSubcoreMesh(core_axis_name="c", subcore_axis_name="s",
                                  num_cores=NC, num_subcores=NS)

    @functools.partial(
        pl.kernel,
        mesh=mesh,
        out_type=jax.ShapeDtypeStruct((B, 2 * D), jnp.float32),
        scratch_types=[
            pltpu.VMEM((b_per_w,), jnp.int32),
        ] + [pltpu.VMEM((C, 2 * D), jnp.float32) for _ in range(_NBUF)]
          + [pltpu.SemaphoreType.DMA for _ in range(2 * _NBUF)],
    )
    def emb(idx_hbm, table_hbm, out_hbm, idx_v, *rest):
        bufs = rest[:_NBUF]
        sem_g = rest[_NBUF:2 * _NBUF]
        sem_w = rest[2 * _NBUF:]
        wid = lax.axis_index("s") * NC + lax.axis_index("c")
        base = wid * b_per_w

        pltpu.sync_copy(idx_hbm.at[pl.ds(base, b_per_w)], idx_v)

        def gather_start(i, b):
            pltpu.async_copy(table_hbm.at[idx_v.at[pl.ds(i * C, C)]],
                             bufs[b], sem_g[b])

        def gather_wait(b):
            pltpu.make_async_copy(table_hbm.at[idx_v.at[pl.ds(0, C)]],
                                  bufs[b], sem_g[b]).wait()

        def wb_start(i, b):
            pltpu.async_copy(bufs[b],
                             out_hbm.at[pl.ds(base + i * C, C)], sem_w[b])

        def wb_wait(b):
            pltpu.make_async_copy(bufs[b],
                                  out_hbm.at[pl.ds(base, C)], sem_w[b]).wait()

        for b in range(_NBUF):
            gather_start(b, b)

        def outer(g, carry):
            for b in range(_NBUF):
                i = g * _NBUF + b
                gather_wait(b)
                wb_start(i, b)
                wb_wait(b)
                gather_start(i + _NBUF, b)
            return carry

        lax.fori_loop(0, n_outer - 1, outer, 0)

        for b in range(_NBUF):
            i = (n_outer - 1) * _NBUF + b
            gather_wait(b)
            wb_start(i, b)
            wb_wait(b)

    return emb


def kernel(x, table):
    Bx, S = x.shape
    V, D = table.shape
    B = Bx * S
    info = plsc.get_sparse_core_info()
    idx = x.reshape(B).astype(jnp.int32)
    tpad = jnp.pad(table, ((0, 0), (0, D)))
    emb = _emb_call(B, V, D, info.num_cores, info.num_subcores, C=400)
    out = emb(idx, tpad)
    return out.reshape(Bx, S, 2 * D)[..., :D]


# speedup vs baseline: 1.3339x; 1.0884x over previous
import functools

import jax
import jax.numpy as jnp
from jax import lax
from jax.experimental import pallas as pl
from jax.experimental.pallas import tpu as pltpu
from jax.experimental.pallas import tpu_sc as plsc

_NBUF = 2


def _emb_call(B, V, D, NC, NS, C):
    NW = NC * NS
    b_per_w = B // NW
    n_chunks = b_per_w // C
    n_outer = n_chunks // _NBUF
    mesh = plsc.VectorSubcoreMesh(core_axis_name="c", subcore_axis_name="s",
                                  num_cores=NC, num_subcores=NS)

    @functools.partial(
        pl.kernel,
        mesh=mesh,
        out_type=jax.ShapeDtypeStruct((B, 2 * D), jnp.float32),
        scratch_types=[
            pltpu.VMEM((b_per_w,), jnp.int32),
        ] + [pltpu.VMEM((C, D), jnp.float32) for _ in range(_NBUF)]
          + [pltpu.SemaphoreType.DMA for _ in range(2 * _NBUF)],
        compiler_params=pltpu.CompilerParams(use_tc_tiling_on_sc=False),
    )
    def emb(idx_hbm, table_hbm, out_hbm, idx_v, *rest):
        bufs = rest[:_NBUF]
        sem_g = rest[_NBUF:2 * _NBUF]
        sem_w = rest[2 * _NBUF:]
        wid = lax.axis_index("s") * NC + lax.axis_index("c")
        base = wid * b_per_w

        pltpu.sync_copy(idx_hbm.at[pl.ds(base, b_per_w)], idx_v)

        def gather_start(i, b):
            pltpu.async_copy(table_hbm.at[idx_v.at[pl.ds(i * C, C)]],
                             bufs[b], sem_g[b])

        def gather_wait(b):
            pltpu.make_async_copy(table_hbm.at[idx_v.at[pl.ds(0, C)]],
                                  bufs[b], sem_g[b]).wait()

        def wb_start(i, b):
            pltpu.async_copy(bufs[b],
                             out_hbm.at[pl.ds(base + i * C, C), pl.ds(0, D)],
                             sem_w[b])

        def wb_wait(b):
            pltpu.make_async_copy(bufs[b],
                                  out_hbm.at[pl.ds(base, C), pl.ds(0, D)],
                                  sem_w[b]).wait()

        for b in range(_NBUF):
            gather_start(b, b)

        def outer(g, carry):
            for b in range(_NBUF):
                i = g * _NBUF + b
                gather_wait(b)
                wb_start(i, b)
                wb_wait(b)
                gather_start(i + _NBUF, b)
            return carry

        lax.fori_loop(0, n_outer - 1, outer, 0)

        for b in range(_NBUF):
            i = (n_outer - 1) * _NBUF + b
            gather_wait(b)
            wb_start(i, b)
            wb_wait(b)

    return emb


def kernel(x, table):
    Bx, S = x.shape
    V, D = table.shape
    B = Bx * S
    info = plsc.get_sparse_core_info()
    idx = x.reshape(B).astype(jnp.int32)
    emb = _emb_call(B, V, D, info.num_cores, info.num_subcores, C=400)
    out = emb(idx, table)
    return out.reshape(Bx, S, 2 * D)[..., :D]
